# baseline (speedup 1.0000x reference)
import functools

import jax
import jax.numpy as jnp
from jax import lax
from jax.experimental import pallas as pl
from jax.experimental.pallas import tpu as pltpu
from jax.experimental.pallas import tpu_sc as plsc

_B = 4
_H = 1
_L = 4096
_DK = 128
_S = 512
_N = _B * _H * _L
_GAMMA = 0.99

_BLK = 8192
_NH = _N // _BLK

_NC = 2
_NS = 16
_NW = _NC * _NS
_TPW = _BLK // _NW
_CH = 128
_NCH = _TPW // _CH


def _dist_body(v_ref, lm_ref, csum_ref, ccnt_ref,
               z_ref, e_ref, c_ref, lcm_ref, lcb_ref, s2_ref):
    cn = ccnt_ref[0, :]
    cc = jnp.clip(cn, 0.01, None)
    c = csum_ref[...] / cc[:, None]
    v = v_ref[...]
    scores = lax.dot_general(
        v, c, dimension_numbers=(((1,), (1,)), ((), ())),
        preferred_element_type=jnp.float32)
    vsq = jnp.sum(v * v, axis=1, keepdims=True)
    cbsq = jnp.sum(c * c, axis=1)
    d = vsq - 2.0 * scores + cbsq[None, :]
    m = jnp.min(d, axis=1, keepdims=True)
    eq = d == m
    z = jnp.argmin(d, axis=1).astype(jnp.int32)
    e = jnp.maximum(m[:, 0], 0.0)
    z_ref[0, 0, :] = z
    e_ref[0, 0, :] = e
    c_ref[...] = c

    lm = lm_ref[0, 0, :].astype(jnp.float32)
    oh = jnp.where(eq, 1.0, 0.0)
    u = jnp.stack([lm, lm * (vsq[:, 0] - m[:, 0]) * 0.5], axis=0)
    c2 = lax.dot_general(
        u, oh, dimension_numbers=(((1,), (0,)), ((), ())),
        preferred_element_type=jnp.float32)
    g = 0.5 * (cc * cbsq) + cn
    lcb_ref[0, 0] = jnp.sum(c2[0, :] * g + c2[1, :] * cc)
    lcm_ref[0, 0] = jnp.sum(e * lm)
    cs = csum_ref[...]
    s2_ref[0, 0] = jnp.sum(cs * cs) + jnp.sum(cn * cn)


def _make_dist_call(h):
    return pl.pallas_call(
        _dist_body,
        grid=(1,),
        in_specs=[
            pl.BlockSpec((_BLK, _DK), lambda i, h=h: (h, 0)),
            pl.BlockSpec((1, 1, _BLK), lambda i, h=h: (h, 0, 0)),
            pl.BlockSpec((_S, _DK), lambda i: (0, 0)),
            pl.BlockSpec((1, _S), lambda i: (0, 0)),
        ],
        out_specs=[
            pl.BlockSpec((1, 1, _BLK), lambda i: (0, 0, 0)),
            pl.BlockSpec((1, 1, _BLK), lambda i: (0, 0, 0)),
            pl.BlockSpec((_S, _DK), lambda i: (0, 0)),
            pl.BlockSpec(memory_space=pltpu.SMEM),
            pl.BlockSpec(memory_space=pltpu.SMEM),
            pl.BlockSpec(memory_space=pltpu.SMEM),
        ],
        out_shape=[
            jax.ShapeDtypeStruct((1, 1, _BLK), jnp.int32),
            jax.ShapeDtypeStruct((1, 1, _BLK), jnp.float32),
            jax.ShapeDtypeStruct((_S, _DK), jnp.float32),
            jax.ShapeDtypeStruct((1, 1), jnp.float32),
            jax.ShapeDtypeStruct((1, 1), jnp.float32),
            jax.ShapeDtypeStruct((1, 1), jnp.float32),
        ],
    )


_dist_calls = [_make_dist_call(h) for h in range(_NH)]


def _sc_body(c_hbm, z_hbm, vh_hbm, idxall, rows, sem):
    cid = lax.axis_index("c")
    sid = lax.axis_index("s")
    wid = sid * _NC + cid
    base = wid * _TPW
    pltpu.sync_copy(z_hbm.at[pl.ds(base, _TPW)], idxall)
    gathers = []
    for k in range(_NCH):
        gathers.append(pltpu.async_copy(
            c_hbm.at[idxall.at[pl.ds(k * _CH, _CH)]],
            rows.at[pl.ds(k * _CH, _CH)], sem))
    for g in gathers:
        g.wait()
    pltpu.sync_copy(rows, vh_hbm.at[pl.ds(base, _TPW)])


@functools.lru_cache(maxsize=1)
def _get_sc_call():
    return functools.partial(
        pl.kernel,
        mesh=plsc.VectorSubcoreMesh(core_axis_name="c", subcore_axis_name="s"),
        out_type=jax.ShapeDtypeStruct((_BLK, _DK), jnp.float32),
        scratch_types=[
            pltpu.VMEM((_TPW,), jnp.int32),
            pltpu.VMEM((_TPW, _DK), jnp.float32),
            pltpu.SemaphoreType.DMA,
        ],
    )(_sc_body)


def kernel(vecs, loss_mask, c_sum, c_count, n_device, n_block_per_update):
    v2 = vecs.reshape(_N, _DK)
    lm3 = loss_mask.reshape(_NH, 1, _BLK)
    cs2 = c_sum.reshape(_S, _DK)
    cc2 = c_count.reshape(1, _S)

    zs, es, lcms, lcbs, vhs = [], [], [], [], []
    c = None
    s2 = None
    sc_call = _get_sc_call()
    for h in range(_NH):
        z3, e3, ch, lcm, lcb, s2h = _dist_calls[h](v2, lm3, cs2, cc2)
        if c is None:
            c, s2 = ch, s2h
        zs.append(z3)
        es.append(e3)
        lcms.append(lcm[0, 0])
        lcbs.append(lcb[0, 0])
    for h in range(_NH):
        vhs.append(sc_call(c, zs[h].reshape(_BLK)))

    scale = (jnp.asarray(n_device, jnp.float32)
             * jnp.asarray(n_block_per_update, jnp.float32))
    l_commit = sum(lcms) * (1.0 / float(_N))
    l_codebook = (1.0 - _GAMMA) * (s2[0, 0] - scale * sum(lcbs))

    vecs_hat = jnp.concatenate(vhs, axis=0).reshape(_B, _H, _L, _DK)
    z = jnp.concatenate(
        [a.reshape(_BLK) for a in zs], axis=0).reshape(_B, _H, _L)
    errs2 = jnp.concatenate(
        [a.reshape(_BLK) for a in es], axis=0).reshape(_B, _H, _L)
    return vecs_hat, z, l_commit, l_codebook, errs2


# speedup vs baseline: 1.2188x; 1.2188x over previous
import functools

import jax
import jax.numpy as jnp
from jax import lax
from jax.experimental import pallas as pl
from jax.experimental.pallas import tpu as pltpu
from jax.experimental.pallas import tpu_sc as plsc

_B = 4
_H = 1
_L = 4096
_DK = 128
_S = 512
_N = _B * _H * _L
_GAMMA = 0.99

_BLK = 8192
_NB = _N // _BLK

_NC = 2
_NS = 16
_NW = _NC * _NS
_TPW = _N // _NW
_CH = 128
_NCH = _TPW // _CH


def _dist_body(v_ref, lm_ref, csum_ref, ccnt_ref, scale_ref,
               z_ref, e_ref, c_ref, lcm_ref, lcb_ref):
    i = pl.program_id(0)
    cn = ccnt_ref[0, :]
    cc = jnp.clip(cn, 0.01, None)
    c = csum_ref[...] / cc[:, None]
    v = v_ref[...]
    scores = lax.dot_general(
        v, c, dimension_numbers=(((1,), (1,)), ((), ())),
        preferred_element_type=jnp.float32)
    vsq = jnp.sum(v * v, axis=1, keepdims=True)
    cbsq = jnp.sum(c * c, axis=1)
    d = vsq - 2.0 * scores + cbsq[None, :]
    m = jnp.min(d, axis=1, keepdims=True)
    eq = d == m
    z = jnp.argmin(d, axis=1).astype(jnp.int32)
    e = jnp.maximum(m[:, 0], 0.0)
    z_ref[0, 0, :] = z
    e_ref[0, 0, :] = e

    lm = lm_ref[0, 0, :].astype(jnp.float32)
    oh = jnp.where(eq, 1.0, 0.0)
    u = jnp.stack([lm, lm * (vsq[:, 0] - m[:, 0]) * 0.5], axis=0)
    c2 = lax.dot_general(
        u, oh, dimension_numbers=(((1,), (0,)), ((), ())),
        preferred_element_type=jnp.float32)
    g = 0.5 * (cc * cbsq) + cn
    part_cb = jnp.sum(c2[0, :] * g + c2[1, :] * cc)
    part_commit = jnp.sum(e * lm)

    @pl.when(i == 0)
    def _():
        c_ref[...] = c
        lcm_ref[0, 0] = 0.0
        lcb_ref[0, 0] = 0.0

    lcm_ref[0, 0] += part_commit
    lcb_ref[0, 0] += part_cb

    @pl.when(i == _NB - 1)
    def _():
        cs = csum_ref[...]
        s2 = jnp.sum(cs * cs) + jnp.sum(cn * cn)
        sc = scale_ref[0, 0]
        lcb_ref[0, 0] = (1.0 - _GAMMA) * (s2 - sc * lcb_ref[0, 0])
        lcm_ref[0, 0] = lcm_ref[0, 0] * (1.0 / float(_N))


_dist_call = pl.pallas_call(
    _dist_body,
    grid=(_NB,),
    in_specs=[
        pl.BlockSpec((_BLK, _DK), lambda i: (i, 0)),
        pl.BlockSpec((1, 1, _BLK), lambda i: (i, 0, 0)),
        pl.BlockSpec((_S, _DK), lambda i: (0, 0)),
        pl.BlockSpec((1, _S), lambda i: (0, 0)),
        pl.BlockSpec(memory_space=pltpu.SMEM),
    ],
    out_specs=[
        pl.BlockSpec((1, 1, _BLK), lambda i: (i, 0, 0)),
        pl.BlockSpec((1, 1, _BLK), lambda i: (i, 0, 0)),
        pl.BlockSpec((_S, _DK), lambda i: (0, 0)),
        pl.BlockSpec(memory_space=pltpu.SMEM),
        pl.BlockSpec(memory_space=pltpu.SMEM),
    ],
    out_shape=[
        jax.ShapeDtypeStruct((_NB, 1, _BLK), jnp.int32),
        jax.ShapeDtypeStruct((_NB, 1, _BLK), jnp.float32),
        jax.ShapeDtypeStruct((_S, _DK), jnp.float32),
        jax.ShapeDtypeStruct((1, 1), jnp.float32),
        jax.ShapeDtypeStruct((1, 1), jnp.float32),
    ],
    compiler_params=pltpu.CompilerParams(
        dimension_semantics=("arbitrary",)),
)


def _sc_body(c_hbm, z_hbm, vh_hbm, idxall, rows, sem):
    cid = lax.axis_index("c")
    sid = lax.axis_index("s")
    wid = sid * _NC + cid
    base = wid * _TPW
    pltpu.sync_copy(z_hbm.at[pl.ds(base, _TPW)], idxall)
    gathers = []
    for k in range(_NCH):
        gathers.append(pltpu.async_copy(
            c_hbm.at[idxall.at[pl.ds(k * _CH, _CH)]],
            rows.at[pl.ds(k * _CH, _CH)], sem))
    for g in gathers:
        g.wait()
    pltpu.sync_copy(rows, vh_hbm.at[pl.ds(base, _TPW)])


@functools.lru_cache(maxsize=1)
def _get_sc_call():
    return functools.partial(
        pl.kernel,
        mesh=plsc.VectorSubcoreMesh(core_axis_name="c", subcore_axis_name="s"),
        out_type=jax.ShapeDtypeStruct((_N, _DK), jnp.float32),
        scratch_types=[
            pltpu.VMEM((_TPW,), jnp.int32),
            pltpu.VMEM((_TPW, _DK), jnp.float32),
            pltpu.SemaphoreType.DMA,
        ],
    )(_sc_body)


def kernel(vecs, loss_mask, c_sum, c_count, n_device, n_block_per_update):
    v2 = vecs.reshape(_N, _DK)
    lm3 = loss_mask.reshape(_NB, 1, _BLK)
    cs2 = c_sum.reshape(_S, _DK)
    cc2 = c_count.reshape(1, _S)
    scale = (jnp.asarray(n_device, jnp.float32)
             * jnp.asarray(n_block_per_update, jnp.float32)).reshape(1, 1)

    z3, e3, c, lcm, lcb = _dist_call(v2, lm3, cs2, cc2, scale)
    vh = _get_sc_call()(c, z3.reshape(_N))

    vecs_hat = vh.reshape(_B, _H, _L, _DK)
    z = z3.reshape(_B, _H, _L)
    errs2 = e3.reshape(_B, _H, _L)
    return vecs_hat, z, lcm[0, 0], lcb[0, 0], errs2
